# baseline (speedup 1.0000x reference)
import functools

import jax
import jax.numpy as jnp
from jax import lax
from jax.experimental import pallas as pl
from jax.experimental.pallas import tpu as pltpu
from jax.experimental.pallas import tpu_sc as plsc

_N = 10000
_E = 160000
_H = 512
_G = 64

_C_EDGES = 80
_EPT = _E // 16
_NCHUNK = _EPT // _C_EDGES
_ZROWS = 40
_NZCHUNK = _N // _ZROWS


def _linear4(x, W, b, rows, row_block, k_dim):
    grid = rows // row_block

    def body(x_ref, w_ref, b_ref, out_ref):
        h = jnp.dot(x_ref[...], w_ref[...],
                    preferred_element_type=jnp.float32) + b_ref[...]
        for q in range(4):
            out_ref[q] = h[:, q * 128:(q + 1) * 128]

    return pl.pallas_call(
        body,
        grid=(grid,),
        in_specs=[
            pl.BlockSpec((row_block, k_dim), lambda r: (r, 0)),
            pl.BlockSpec((k_dim, _H), lambda r: (0, 0)),
            pl.BlockSpec((1, _H), lambda r: (0, 0)),
        ],
        out_specs=pl.BlockSpec((4, row_block, 128), lambda r: (0, r, 0)),
        out_shape=jax.ShapeDtypeStruct((4, rows, 128), jnp.float32),
    )(x, W, b.reshape(1, _H))


def _edge_linear_packed(x, W, b):
    row_block = 8000
    grid = _E // row_block

    def body(x_ref, w_ref, b_ref, out_ref):
        h = jnp.dot(x_ref[...], w_ref[...],
                    preferred_element_type=jnp.float32) + b_ref[...]
        hb = h.astype(jnp.bfloat16).reshape(row_block // 2, 2 * _H)
        lo = jax.lax.bitcast_convert_type(hb[:, :_H],
                                          jnp.uint16).astype(jnp.uint32)
        hi = jax.lax.bitcast_convert_type(hb[:, _H:],
                                          jnp.uint16).astype(jnp.uint32)
        packed = (lo | (hi << 16)).astype(jnp.int32)
        for q in range(4):
            out_ref[q] = packed[:, q * 128:(q + 1) * 128]

    return pl.pallas_call(
        body,
        grid=(grid,),
        in_specs=[
            pl.BlockSpec((row_block, 16), lambda r: (r, 0)),
            pl.BlockSpec((16, _H), lambda r: (0, 0)),
            pl.BlockSpec((1, _H), lambda r: (0, 0)),
        ],
        out_specs=pl.BlockSpec((4, row_block // 2, 128), lambda r: (0, r, 0)),
        out_shape=jax.ShapeDtypeStruct((4, _E // 2, 128), jnp.int32),
    )(x, W, b.reshape(1, _H))


def _sc_edge_aggregate(h4, e4, src, dst):
    mesh = plsc.VectorSubcoreMesh(core_axis_name="c", subcore_axis_name="s")

    @functools.partial(
        pl.kernel,
        mesh=mesh,
        out_type=jax.ShapeDtypeStruct((4 * _N, 128), jnp.float32),
        scratch_types=[
            pltpu.VMEM((_C_EDGES,), jnp.int32),
            pltpu.VMEM((_C_EDGES,), jnp.int32),
            pltpu.VMEM((_C_EDGES,), jnp.int32),
            pltpu.VMEM((_C_EDGES,), jnp.int32),
            pltpu.VMEM((_C_EDGES, 128), jnp.float32),
            pltpu.VMEM((_C_EDGES, 128), jnp.float32),
            pltpu.VMEM((_C_EDGES // 2, 128), jnp.int32),
            pltpu.VMEM((_C_EDGES // 2, 128), jnp.int32),
            pltpu.VMEM((_ZROWS, 128), jnp.float32),
            pltpu.VMEM_SHARED((_N, 128), jnp.float32),
            pltpu.SemaphoreType.DMA,
            pltpu.SemaphoreType.DMA,
            pltpu.SemaphoreType.DMA,
            pltpu.SemaphoreType.DMA,
            pltpu.SemaphoreType.DMA,
        ],
    )
    def k(h_hbm, e_hbm, src_hbm, dst_hbm, out_hbm,
          src_v0, src_v1, dst_v0, dst_v1, rows_v0, rows_v1, e_v0, e_v1,
          zero_v, aggr_sh, sem_idx, semg0, semg1, seme0, seme1):
        cid = lax.axis_index("c")
        sid = lax.axis_index("s")
        ebase = sid * _EPT
        src_v = (src_v0, src_v1)
        dst_v = (dst_v0, dst_v1)
        rows_v = (rows_v0, rows_v1)
        e_v = (e_v0, e_v1)
        semg = (semg0, semg1)
        seme = (seme0, seme1)

        def zbody(j, carry):
            for t in range(8):
                zero_v[j, pl.ds(t * 16, 16)] = jnp.zeros((16,), jnp.float32)
            return carry

        lax.fori_loop(0, _ZROWS, zbody, 0)

        def front(kk, b, q):
            base = ebase + kk * _C_EDGES
            cp1 = pltpu.async_copy(src_hbm.at[pl.ds(base, _C_EDGES)],
                                   src_v[b], sem_idx)
            cp2 = pltpu.async_copy(dst_hbm.at[pl.ds(base, _C_EDGES)],
                                   dst_v[b], sem_idx)
            cp1.wait()
            cp2.wait()
            offs = q * _N

            def obody(t, c2):
                sl = pl.ds(t * 16, 16)
                src_v[b][sl] = src_v[b][sl] + offs
                return c2

            lax.fori_loop(0, _C_EDGES // 16, obody, 0)
            pltpu.async_copy(h_hbm.at[src_v[b]], rows_v[b], semg[b])
            ebase2 = (q * (_E // 2) + sid * (_EPT // 2)
                      + kk * (_C_EDGES // 2))
            pltpu.async_copy(
                e_hbm.at[pl.ds(ebase2, _C_EDGES // 2)],
                e_v[b], seme[b])

        def back(b):
            pltpu.make_async_copy(h_hbm.at[pl.ds(0, _C_EDGES)],
                                  rows_v[b], semg[b]).wait()
            pltpu.make_async_copy(e_hbm.at[pl.ds(0, _C_EDGES // 2)],
                                  e_v[b], seme[b]).wait()
            hi_mask = jnp.full((16,), -65536, jnp.int32)

            def cbody(j, c2):
                for t in range(8):
                    sl = pl.ds(t * 16, 16)
                    w = e_v[b][j, sl]
                    elo = lax.bitcast_convert_type(w << 16, jnp.float32)
                    ehi = lax.bitcast_convert_type(w & hi_mask, jnp.float32)
                    rows_v[b][2 * j, sl] = jnp.maximum(
                        rows_v[b][2 * j, sl] + elo, 0.0)
                    rows_v[b][2 * j + 1, sl] = jnp.maximum(
                        rows_v[b][2 * j + 1, sl] + ehi, 0.0)
                return c2

            lax.fori_loop(0, _C_EDGES // 2, cbody, 0)
            pltpu.sync_copy(rows_v[b], aggr_sh.at[dst_v[b]], add=True)

        for qi in range(2):
            q = 2 * cid + qi
            for m in range((_NZCHUNK + 15) // 16):
                ci = sid + 16 * m

                @pl.when(ci < _NZCHUNK)
                def _():
                    pltpu.sync_copy(zero_v,
                                    aggr_sh.at[pl.ds(ci * _ZROWS, _ZROWS)])
            plsc.subcore_barrier()

            front(0, 0, q)
            front(1, 1, q)

            def pair(i, carry):
                kk = 2 * i
                back(0)
                front(kk + 2, 0, q)
                back(1)

                @pl.when(i < _NCHUNK // 2 - 1)
                def _():
                    front(kk + 3, 1, q)
                return carry

            lax.fori_loop(0, _NCHUNK // 2, pair, 0)
            back(0)
            plsc.subcore_barrier()
            for m in range((_NZCHUNK + 15) // 16):
                ci = sid + 16 * m

                @pl.when(ci < _NZCHUNK)
                def _():
                    r0 = ci * _ZROWS
                    pltpu.sync_copy(aggr_sh.at[pl.ds(r0, _ZROWS)],
                                    out_hbm.at[pl.ds(q * _N + r0, _ZROWS)])
            plsc.subcore_barrier()

    return k(h4, e4, src, dst)


def _mlp_layer(h4, aggr4, alpha, W1i, b1i, W2i, b2i, g, be, mu, var):
    row_block = 1000
    grid = _N // row_block

    def body(al_ref, h_ref, a_ref, w1_ref, b1_ref, w2_ref, b2_ref,
             g_ref, be_ref, mu_ref, var_ref, out_ref):
        hcat = jnp.concatenate([h_ref[q] for q in range(4)], axis=1)
        acat = jnp.concatenate([a_ref[q] for q in range(4)], axis=1)
        z = hcat * al_ref[:, 0:1] + acat
        t = jnp.maximum(
            jnp.dot(z, w1_ref[...], preferred_element_type=jnp.float32)
            + b1_ref[...], 0.0)
        o = jnp.dot(t, w2_ref[...],
                    preferred_element_type=jnp.float32) + b2_ref[...]
        scale = g_ref[...] * lax.rsqrt(var_ref[...] + 1e-5)
        o = (o - mu_ref[...]) * scale + be_ref[...]
        o = jnp.maximum(o, 0.0)
        for q in range(4):
            out_ref[q] = o[:, q * 128:(q + 1) * 128]

    return pl.pallas_call(
        body,
        grid=(grid,),
        in_specs=[
            pl.BlockSpec((1, 128), lambda r: (0, 0)),
            pl.BlockSpec((4, row_block, 128), lambda r: (0, r, 0)),
            pl.BlockSpec((4, row_block, 128), lambda r: (0, r, 0)),
            pl.BlockSpec((_H, 2 * _H), lambda r: (0, 0)),
            pl.BlockSpec((1, 2 * _H), lambda r: (0, 0)),
            pl.BlockSpec((2 * _H, _H), lambda r: (0, 0)),
            pl.BlockSpec((1, _H), lambda r: (0, 0)),
            pl.BlockSpec((1, _H), lambda r: (0, 0)),
            pl.BlockSpec((1, _H), lambda r: (0, 0)),
            pl.BlockSpec((1, _H), lambda r: (0, 0)),
            pl.BlockSpec((1, _H), lambda r: (0, 0)),
        ],
        out_specs=pl.BlockSpec((4, row_block, 128), lambda r: (0, r, 0)),
        out_shape=jax.ShapeDtypeStruct((4, _N, 128), jnp.float32),
    )(alpha, h4, aggr4, W1i, b1i.reshape(1, 2 * _H), W2i,
      b2i.reshape(1, _H), g.reshape(1, _H), be.reshape(1, _H),
      mu.reshape(1, _H), var.reshape(1, _H))


def _pool(h4, batch2d):
    row_block = 1000
    grid = _N // row_block

    def body(h_ref, b_ref, sums_ref, counts_ref):
        r = pl.program_id(0)
        hcat = jnp.concatenate([h_ref[q] for q in range(4)], axis=1)
        onehot = (b_ref[...] == lax.broadcasted_iota(
            jnp.int32, (1, _G), 1)).astype(jnp.float32)
        s = lax.dot_general(onehot, hcat, (((0,), (0,)), ((), ())),
                            preferred_element_type=jnp.float32)
        c = lax.dot_general(onehot, jnp.ones((row_block, 128), jnp.float32),
                            (((0,), (0,)), ((), ())),
                            preferred_element_type=jnp.float32)

        @pl.when(r == 0)
        def _():
            sums_ref[...] = jnp.zeros_like(sums_ref)
            counts_ref[...] = jnp.zeros_like(counts_ref)

        sums_ref[...] += s
        counts_ref[...] += c

    return pl.pallas_call(
        body,
        grid=(grid,),
        in_specs=[
            pl.BlockSpec((4, row_block, 128), lambda r: (0, r, 0)),
            pl.BlockSpec((row_block, 1), lambda r: (r, 0)),
        ],
        out_specs=[
            pl.BlockSpec((_G, _H), lambda r: (0, 0)),
            pl.BlockSpec((_G, 128), lambda r: (0, 0)),
        ],
        out_shape=[
            jax.ShapeDtypeStruct((_G, _H), jnp.float32),
            jax.ShapeDtypeStruct((_G, 128), jnp.float32),
        ],
    )(h4, batch2d)


def _head(sums, counts, Wm1, bm1, Wm2b, bm2b):
    def body(s_ref, c_ref, w1_ref, b1_ref, w2_ref, b2_ref, out_ref):
        cnt = jnp.maximum(c_ref[:, 0:1], 1.0)
        pooled = s_ref[...] / cnt
        t = jnp.maximum(
            jnp.dot(pooled, w1_ref[...], preferred_element_type=jnp.float32)
            + b1_ref[...], 0.0)
        out_ref[...] = jnp.dot(
            t, w2_ref[...], preferred_element_type=jnp.float32) + b2_ref[...]

    return pl.pallas_call(
        body,
        out_shape=jax.ShapeDtypeStruct((_G, 128), jnp.float32),
    )(sums, counts, Wm1, bm1.reshape(1, _H // 2), Wm2b, bm2b)


def kernel(x, edge_attr, edge_index, batch, W_node, b_node, W_edge, b_edge,
           eps, W1, b1, W2, b2, bn_gamma, bn_beta, bn_mean, bn_var,
           Wm1, bm1, Wm2, bm2):
    src = edge_index[0].astype(jnp.int32)
    dst = edge_index[1].astype(jnp.int32)

    h4 = _linear4(x, W_node, b_node, _N, 1000, 256)
    e4 = _edge_linear_packed(edge_attr, W_edge, b_edge)
    e4r = e4.reshape(2 * _E, 128)

    for i in range(3):
        aggr4 = _sc_edge_aggregate(
            h4.reshape(4 * _N, 128), e4r, src, dst).reshape(4, _N, 128)
        alpha = jnp.full((1, 128), 1.0 + eps[i], jnp.float32)
        h4 = _mlp_layer(h4, aggr4, alpha, W1[i], b1[i], W2[i], b2[i],
                        bn_gamma[i], bn_beta[i], bn_mean[i], bn_var[i])

    sums, counts = _pool(h4, batch.astype(jnp.int32).reshape(_N, 1))
    Wm2b = jnp.broadcast_to(Wm2, (_H // 2, 128))
    bm2b = jnp.broadcast_to(bm2.reshape(1, 1), (1, 128))
    out = _head(sums, counts, Wm1, bm1, Wm2b, bm2b)
    return out[:, :1]


# speedup vs baseline: 1.8380x; 1.8380x over previous
import functools

import jax
import jax.numpy as jnp
from jax import lax
from jax.experimental import pallas as pl
from jax.experimental.pallas import tpu as pltpu
from jax.experimental.pallas import tpu_sc as plsc

_N = 10000
_E = 160000
_H = 512
_G = 64

_C_EDGES = 80
_EPT = _E // 16
_NCHUNK = _EPT // _C_EDGES
_ZROWS = 40
_NZCHUNK = _N // _ZROWS


def _linear4(x, W, b, rows, row_block, k_dim):
    grid = rows // row_block

    def body(x_ref, w_ref, b_ref, out_ref):
        h = jnp.dot(x_ref[...], w_ref[...],
                    preferred_element_type=jnp.float32) + b_ref[...]
        for q in range(4):
            out_ref[q] = h[:, q * 128:(q + 1) * 128]

    return pl.pallas_call(
        body,
        grid=(grid,),
        in_specs=[
            pl.BlockSpec((row_block, k_dim), lambda r: (r, 0)),
            pl.BlockSpec((k_dim, _H), lambda r: (0, 0)),
            pl.BlockSpec((1, _H), lambda r: (0, 0)),
        ],
        out_specs=pl.BlockSpec((4, row_block, 128), lambda r: (0, r, 0)),
        out_shape=jax.ShapeDtypeStruct((4, rows, 128), jnp.float32),
    )(x, W, b.reshape(1, _H))


def _edge_linear_packed(x, W, b):
    row_block = 8000
    grid = _E // row_block

    def body(x_ref, w_ref, b_ref, out_ref):
        h = jnp.dot(x_ref[...], w_ref[...],
                    preferred_element_type=jnp.float32) + b_ref[...]
        hb = h.astype(jnp.bfloat16).reshape(row_block // 2, 2 * _H)
        lo = jax.lax.bitcast_convert_type(hb[:, :_H],
                                          jnp.uint16).astype(jnp.uint32)
        hi = jax.lax.bitcast_convert_type(hb[:, _H:],
                                          jnp.uint16).astype(jnp.uint32)
        packed = (lo | (hi << 16)).astype(jnp.int32)
        for q in range(4):
            out_ref[q] = packed[:, q * 128:(q + 1) * 128]

    return pl.pallas_call(
        body,
        grid=(grid,),
        in_specs=[
            pl.BlockSpec((row_block, 16), lambda r: (r, 0)),
            pl.BlockSpec((16, _H), lambda r: (0, 0)),
            pl.BlockSpec((1, _H), lambda r: (0, 0)),
        ],
        out_specs=pl.BlockSpec((4, row_block // 2, 128), lambda r: (0, r, 0)),
        out_shape=jax.ShapeDtypeStruct((4, _E // 2, 128), jnp.int32),
    )(x, W, b.reshape(1, _H))


def _sc_edge_aggregate(h4, e4, src, dst):
    mesh = plsc.VectorSubcoreMesh(core_axis_name="c", subcore_axis_name="s")

    @functools.partial(
        pl.kernel,
        mesh=mesh,
        out_type=jax.ShapeDtypeStruct((4 * _N, 128), jnp.float32),
        scratch_types=[
            pltpu.VMEM((_C_EDGES,), jnp.int32),
            pltpu.VMEM((_C_EDGES,), jnp.int32),
            pltpu.VMEM((_C_EDGES,), jnp.int32),
            pltpu.VMEM((_C_EDGES,), jnp.int32),
            pltpu.VMEM((_C_EDGES, 128), jnp.float32),
            pltpu.VMEM((_C_EDGES, 128), jnp.float32),
            pltpu.VMEM((_C_EDGES // 2, 128), jnp.int32),
            pltpu.VMEM((_C_EDGES // 2, 128), jnp.int32),
            pltpu.VMEM((_ZROWS, 128), jnp.float32),
            pltpu.VMEM_SHARED((_N, 128), jnp.float32),
            pltpu.SemaphoreType.DMA,
            pltpu.SemaphoreType.DMA,
            pltpu.SemaphoreType.DMA,
            pltpu.SemaphoreType.DMA,
            pltpu.SemaphoreType.DMA,
        ],
    )
    def k(h_hbm, e_hbm, src_hbm, dst_hbm, out_hbm,
          src_v0, src_v1, dst_v0, dst_v1, rows_v0, rows_v1, e_v0, e_v1,
          zero_v, aggr_sh, sem_idx, semg0, semg1, seme0, seme1):
        cid = lax.axis_index("c")
        sid = lax.axis_index("s")
        ebase = sid * _EPT
        src_v = (src_v0, src_v1)
        dst_v = (dst_v0, dst_v1)
        rows_v = (rows_v0, rows_v1)
        e_v = (e_v0, e_v1)
        semg = (semg0, semg1)
        seme = (seme0, seme1)

        def zbody(j, carry):
            for t in range(8):
                zero_v[j, pl.ds(t * 16, 16)] = jnp.zeros((16,), jnp.float32)
            return carry

        lax.fori_loop(0, _ZROWS, zbody, 0)

        def front(kk, b, q):
            base = ebase + kk * _C_EDGES
            cp1 = pltpu.async_copy(src_hbm.at[pl.ds(base, _C_EDGES)],
                                   src_v[b], sem_idx)
            cp2 = pltpu.async_copy(dst_hbm.at[pl.ds(base, _C_EDGES)],
                                   dst_v[b], sem_idx)
            cp1.wait()
            cp2.wait()
            offs = q * _N

            def obody(t, c2):
                sl = pl.ds(t * 16, 16)
                src_v[b][sl] = src_v[b][sl] + offs
                return c2

            lax.fori_loop(0, _C_EDGES // 16, obody, 0)
            pltpu.async_copy(h_hbm.at[src_v[b]], rows_v[b], semg[b])
            ebase2 = (q * (_E // 2) + sid * (_EPT // 2)
                      + kk * (_C_EDGES // 2))
            pltpu.async_copy(
                e_hbm.at[pl.ds(ebase2, _C_EDGES // 2)],
                e_v[b], seme[b])

        def back(b):
            pltpu.make_async_copy(h_hbm.at[pl.ds(0, _C_EDGES)],
                                  rows_v[b], semg[b]).wait()
            pltpu.make_async_copy(e_hbm.at[pl.ds(0, _C_EDGES // 2)],
                                  e_v[b], seme[b]).wait()
            hi_mask = jnp.full((16,), -65536, jnp.int32)

            @plsc.parallel_loop(0, _C_EDGES // 2, step=1, unroll=4)
            def _(j):
                for t in range(8):
                    sl = pl.ds(t * 16, 16)
                    w = e_v[b][j, sl]
                    elo = lax.bitcast_convert_type(w << 16, jnp.float32)
                    ehi = lax.bitcast_convert_type(w & hi_mask, jnp.float32)
                    rows_v[b][2 * j, sl] = jnp.maximum(
                        rows_v[b][2 * j, sl] + elo, 0.0)
                    rows_v[b][2 * j + 1, sl] = jnp.maximum(
                        rows_v[b][2 * j + 1, sl] + ehi, 0.0)
            pltpu.sync_copy(rows_v[b], aggr_sh.at[dst_v[b]], add=True)

        for qi in range(2):
            q = 2 * cid + qi
            for m in range((_NZCHUNK + 15) // 16):
                ci = sid + 16 * m

                @pl.when(ci < _NZCHUNK)
                def _():
                    pltpu.sync_copy(zero_v,
                                    aggr_sh.at[pl.ds(ci * _ZROWS, _ZROWS)])
            plsc.subcore_barrier()

            front(0, 0, q)
            front(1, 1, q)

            def pair(i, carry):
                kk = 2 * i
                back(0)
                front(kk + 2, 0, q)
                back(1)

                @pl.when(i < _NCHUNK // 2 - 1)
                def _():
                    front(kk + 3, 1, q)
                return carry

            lax.fori_loop(0, _NCHUNK // 2, pair, 0)
            back(0)
            plsc.subcore_barrier()
            for m in range((_NZCHUNK + 15) // 16):
                ci = sid + 16 * m

                @pl.when(ci < _NZCHUNK)
                def _():
                    r0 = ci * _ZROWS
                    pltpu.sync_copy(aggr_sh.at[pl.ds(r0, _ZROWS)],
                                    out_hbm.at[pl.ds(q * _N + r0, _ZROWS)])
            plsc.subcore_barrier()

    return k(h4, e4, src, dst)


def _mlp_layer(h4, aggr4, alpha, W1i, b1i, W2i, b2i, g, be, mu, var):
    row_block = 1000
    grid = _N // row_block

    def body(al_ref, h_ref, a_ref, w1_ref, b1_ref, w2_ref, b2_ref,
             g_ref, be_ref, mu_ref, var_ref, out_ref):
        hcat = jnp.concatenate([h_ref[q] for q in range(4)], axis=1)
        acat = jnp.concatenate([a_ref[q] for q in range(4)], axis=1)
        z = hcat * al_ref[:, 0:1] + acat
        t = jnp.maximum(
            jnp.dot(z, w1_ref[...], preferred_element_type=jnp.float32)
            + b1_ref[...], 0.0)
        o = jnp.dot(t, w2_ref[...],
                    preferred_element_type=jnp.float32) + b2_ref[...]
        scale = g_ref[...] * lax.rsqrt(var_ref[...] + 1e-5)
        o = (o - mu_ref[...]) * scale + be_ref[...]
        o = jnp.maximum(o, 0.0)
        for q in range(4):
            out_ref[q] = o[:, q * 128:(q + 1) * 128]

    return pl.pallas_call(
        body,
        grid=(grid,),
        in_specs=[
            pl.BlockSpec((1, 128), lambda r: (0, 0)),
            pl.BlockSpec((4, row_block, 128), lambda r: (0, r, 0)),
            pl.BlockSpec((4, row_block, 128), lambda r: (0, r, 0)),
            pl.BlockSpec((_H, 2 * _H), lambda r: (0, 0)),
            pl.BlockSpec((1, 2 * _H), lambda r: (0, 0)),
            pl.BlockSpec((2 * _H, _H), lambda r: (0, 0)),
            pl.BlockSpec((1, _H), lambda r: (0, 0)),
            pl.BlockSpec((1, _H), lambda r: (0, 0)),
            pl.BlockSpec((1, _H), lambda r: (0, 0)),
            pl.BlockSpec((1, _H), lambda r: (0, 0)),
            pl.BlockSpec((1, _H), lambda r: (0, 0)),
        ],
        out_specs=pl.BlockSpec((4, row_block, 128), lambda r: (0, r, 0)),
        out_shape=jax.ShapeDtypeStruct((4, _N, 128), jnp.float32),
    )(alpha, h4, aggr4, W1i, b1i.reshape(1, 2 * _H), W2i,
      b2i.reshape(1, _H), g.reshape(1, _H), be.reshape(1, _H),
      mu.reshape(1, _H), var.reshape(1, _H))


def _pool(h4, batch2d):
    row_block = 1000
    grid = _N // row_block

    def body(h_ref, b_ref, sums_ref, counts_ref):
        r = pl.program_id(0)
        hcat = jnp.concatenate([h_ref[q] for q in range(4)], axis=1)
        onehot = (b_ref[...] == lax.broadcasted_iota(
            jnp.int32, (1, _G), 1)).astype(jnp.float32)
        s = lax.dot_general(onehot, hcat, (((0,), (0,)), ((), ())),
                            preferred_element_type=jnp.float32)
        c = lax.dot_general(onehot, jnp.ones((row_block, 128), jnp.float32),
                            (((0,), (0,)), ((), ())),
                            preferred_element_type=jnp.float32)

        @pl.when(r == 0)
        def _():
            sums_ref[...] = jnp.zeros_like(sums_ref)
            counts_ref[...] = jnp.zeros_like(counts_ref)

        sums_ref[...] += s
        counts_ref[...] += c

    return pl.pallas_call(
        body,
        grid=(grid,),
        in_specs=[
            pl.BlockSpec((4, row_block, 128), lambda r: (0, r, 0)),
            pl.BlockSpec((row_block, 1), lambda r: (r, 0)),
        ],
        out_specs=[
            pl.BlockSpec((_G, _H), lambda r: (0, 0)),
            pl.BlockSpec((_G, 128), lambda r: (0, 0)),
        ],
        out_shape=[
            jax.ShapeDtypeStruct((_G, _H), jnp.float32),
            jax.ShapeDtypeStruct((_G, 128), jnp.float32),
        ],
    )(h4, batch2d)


def _head(sums, counts, Wm1, bm1, Wm2b, bm2b):
    def body(s_ref, c_ref, w1_ref, b1_ref, w2_ref, b2_ref, out_ref):
        cnt = jnp.maximum(c_ref[:, 0:1], 1.0)
        pooled = s_ref[...] / cnt
        t = jnp.maximum(
            jnp.dot(pooled, w1_ref[...], preferred_element_type=jnp.float32)
            + b1_ref[...], 0.0)
        out_ref[...] = jnp.dot(
            t, w2_ref[...], preferred_element_type=jnp.float32) + b2_ref[...]

    return pl.pallas_call(
        body,
        out_shape=jax.ShapeDtypeStruct((_G, 128), jnp.float32),
    )(sums, counts, Wm1, bm1.reshape(1, _H // 2), Wm2b, bm2b)


def kernel(x, edge_attr, edge_index, batch, W_node, b_node, W_edge, b_edge,
           eps, W1, b1, W2, b2, bn_gamma, bn_beta, bn_mean, bn_var,
           Wm1, bm1, Wm2, bm2):
    src = edge_index[0].astype(jnp.int32)
    dst = edge_index[1].astype(jnp.int32)

    h4 = _linear4(x, W_node, b_node, _N, 1000, 256)
    e4 = _edge_linear_packed(edge_attr, W_edge, b_edge)
    e4r = e4.reshape(2 * _E, 128)

    for i in range(3):
        aggr4 = _sc_edge_aggregate(
            h4.reshape(4 * _N, 128), e4r, src, dst).reshape(4, _N, 128)
        alpha = jnp.full((1, 128), 1.0 + eps[i], jnp.float32)
        h4 = _mlp_layer(h4, aggr4, alpha, W1[i], b1[i], W2[i], b2[i],
                        bn_gamma[i], bn_beta[i], bn_mean[i], bn_var[i])

    sums, counts = _pool(h4, batch.astype(jnp.int32).reshape(_N, 1))
    Wm2b = jnp.broadcast_to(Wm2, (_H // 2, 128))
    bm2b = jnp.broadcast_to(bm2.reshape(1, 1), (1, 128))
    out = _head(sums, counts, Wm1, bm1, Wm2b, bm2b)
    return out[:, :1]
